# Initial kernel scaffold; baseline (speedup 1.0000x reference)
import functools

import jax
import jax.numpy as jnp
from jax import lax
from jax.experimental import pallas as pl
from jax.experimental.pallas import tpu as pltpu
from jax.experimental.pallas import tpu_sc as plsc

H = 256
IN_GRID = 474
OUT_GRID = 227
RES_H, RES_W = 91, 180
NGD = RES_H * RES_W
NG = 16384
NMD = 2562
NM = 2568
EG = 26624
EM = 20480
EP = 49152
N_PROC = 4

R_MESH = 2592
RD = 8190
R_GRID = RD + 1

BN = 512

_DOT = functools.partial(
    lax.dot_general,
    precision=lax.Precision.HIGHEST,
    preferred_element_type=jnp.float32,
)


def _dot(a, b):
    return _DOT(a, b, dimension_numbers=(((1,), (0,)), ((), ())))


def _dot_t(a, b):
    return _DOT(a, b, dimension_numbers=(((0,), (0,)), ((), ())))


def _ln(y, g, b):
    mu = jnp.mean(y, axis=-1, keepdims=True)
    var = jnp.mean((y - mu) ** 2, axis=-1, keepdims=True)
    return (y - mu) * lax.rsqrt(var + 1e-5) * g + b


def _silu(x):
    return x * jax.nn.sigmoid(x)


def _full(shape):
    nd = len(shape)
    return pl.BlockSpec(shape, lambda i: (0,) * nd)


def _grid_encoder(xt, w1, b1, w2, b2, g, be, wa):
    def body(x_ref, w1_ref, b1_ref, w2_ref, b2_ref, g_ref, be_ref, wa_ref,
             g0_ref, a_ref):
        h = _silu(_dot_t(x_ref[...], w1_ref[...]) + b1_ref[...])
        y = _ln(_dot(h, w2_ref[...]) + b2_ref[...], g_ref[...], be_ref[...])
        g0_ref[...] = y
        a_ref[...] = _dot(y, wa_ref[...])

    return pl.pallas_call(
        body,
        grid=(NG // BN,),
        in_specs=[
            pl.BlockSpec((IN_GRID, BN), lambda i: (0, i)),
            _full((IN_GRID, H)), _full((1, H)), _full((H, H)), _full((1, H)),
            _full((1, H)), _full((1, H)), _full((H, H)),
        ],
        out_specs=[
            pl.BlockSpec((BN, H), lambda i: (i, 0)),
            pl.BlockSpec((BN, H), lambda i: (i, 0)),
        ],
        out_shape=[
            jax.ShapeDtypeStruct((NG, H), jnp.float32),
            jax.ShapeDtypeStruct((NG, H), jnp.float32),
        ],
    )(xt, w1, b1, w2, b2, g, be, wa)


def _mesh_encoder(x, w1, b1, w2, b2, g, be, wb):
    def body(x_ref, w1_ref, b1_ref, w2_ref, b2_ref, g_ref, be_ref, wb_ref,
             m_ref, b_out_ref):
        h = _silu(_dot(x_ref[...], w1_ref[...]) + b1_ref[...])
        y = _ln(_dot(h, w2_ref[...]) + b2_ref[...], g_ref[...], be_ref[...])
        m_ref[...] = y
        b_out_ref[...] = _dot(y, wb_ref[...])

    return pl.pallas_call(
        body,
        out_shape=[
            jax.ShapeDtypeStruct((NM, H), jnp.float32),
            jax.ShapeDtypeStruct((NM, H), jnp.float32),
        ],
    )(x, w1, b1, w2, b2, g, be, wb)


def _edge_encoder(x, w1, b1, w2, b2, g, be):
    E = x.shape[0]

    def body(x_ref, w1_ref, b1_ref, w2_ref, b2_ref, g_ref, be_ref, e_ref):
        h = _silu(_dot(x_ref[...], w1_ref[...]) + b1_ref[...])
        e_ref[...] = _ln(_dot(h, w2_ref[...]) + b2_ref[...],
                         g_ref[...], be_ref[...])

    return pl.pallas_call(
        body,
        grid=(E // BN,),
        in_specs=[
            pl.BlockSpec((BN, 4), lambda i: (i, 0)),
            _full((4, H)), _full((1, H)), _full((H, H)), _full((1, H)),
            _full((1, H)), _full((1, H)),
        ],
        out_specs=[pl.BlockSpec((BN, H), lambda i: (i, 0))],
        out_shape=[jax.ShapeDtypeStruct((E, H), jnp.float32)],
    )(x, w1, b1, w2, b2, g, be)[0]


def _edge_mlp(e, pre0, w1a, b1, w2, b2, g, be):
    E = e.shape[0]

    def body(e_ref, p_ref, w1_ref, b1_ref, w2_ref, b2_ref, g_ref, be_ref,
             ef_ref):
        x = e_ref[...]
        h = _silu(_dot(x, w1_ref[...]) + p_ref[...] + b1_ref[...])
        y = _ln(_dot(h, w2_ref[...]) + b2_ref[...], g_ref[...], be_ref[...])
        ef_ref[...] = y + x

    return pl.pallas_call(
        body,
        grid=(E // BN,),
        in_specs=[
            pl.BlockSpec((BN, H), lambda i: (i, 0)),
            pl.BlockSpec((BN, H), lambda i: (i, 0)),
            _full((H, H)), _full((1, H)), _full((H, H)), _full((1, H)),
            _full((1, H)), _full((1, H)),
        ],
        out_specs=[pl.BlockSpec((BN, H), lambda i: (i, 0))],
        out_shape=[jax.ShapeDtypeStruct((E, H), jnp.float32)],
    )(e, pre0, w1a, b1, w2, b2, g, be)[0]


def _node_mlp_mesh(m, parts, w1a, w1b, b1, w2, b2, g, be, wa, wb):
    def body(m_ref, p_ref, w1a_ref, w1b_ref, b1_ref, w2_ref, b2_ref, g_ref,
             be_ref, wa_ref, wb_ref, m2_ref, a_ref, b_ref):
        x = m_ref[...]
        agg = p_ref[0:NM, :] + p_ref[R_MESH:R_MESH + NM, :]
        h = _silu(_dot(x, w1a_ref[...]) + _dot(agg, w1b_ref[...])
                  + b1_ref[...])
        y = _ln(_dot(h, w2_ref[...]) + b2_ref[...], g_ref[...], be_ref[...])
        m2 = y + x
        m2_ref[...] = m2
        a_ref[...] = _dot(m2, wa_ref[...])
        b_ref[...] = _dot(m2, wb_ref[...])

    return pl.pallas_call(
        body,
        out_shape=[
            jax.ShapeDtypeStruct((NM, H), jnp.float32),
            jax.ShapeDtypeStruct((NM, H), jnp.float32),
            jax.ShapeDtypeStruct((NM, H), jnp.float32),
        ],
    )(m, parts, w1a, w1b, b1, w2, b2, g, be, wa, wb)


def _grid_mlp(g0, w1, b1, w2, b2, g, be, wb):
    def body(x_ref, w1_ref, b1_ref, w2_ref, b2_ref, g_ref, be_ref, wb_ref,
             g1_ref, b_ref):
        x = x_ref[...]
        h = _silu(_dot(x, w1_ref[...]) + b1_ref[...])
        y = _ln(_dot(h, w2_ref[...]) + b2_ref[...], g_ref[...], be_ref[...])
        g1 = y + x
        g1_ref[...] = g1
        b_ref[...] = _dot(g1, wb_ref[...])

    return pl.pallas_call(
        body,
        grid=(NG // BN,),
        in_specs=[
            pl.BlockSpec((BN, H), lambda i: (i, 0)),
            _full((H, H)), _full((1, H)), _full((H, H)), _full((1, H)),
            _full((1, H)), _full((1, H)), _full((H, H)),
        ],
        out_specs=[
            pl.BlockSpec((BN, H), lambda i: (i, 0)),
            pl.BlockSpec((BN, H), lambda i: (i, 0)),
        ],
        out_shape=[
            jax.ShapeDtypeStruct((NG, H), jnp.float32),
            jax.ShapeDtypeStruct((NG, H), jnp.float32),
        ],
    )(g0, w1, b1, w2, b2, g, be, wb)


def _grid_node_decoder(g1, agg, w1a, w1b, b1, w2, b2, g, be,
                       dw1, db1, dw2, db2):
    def body(x_ref, agg_ref, w1a_ref, w1b_ref, b1_ref, w2_ref, b2_ref,
             g_ref, be_ref, dw1_ref, db1_ref, dw2_ref, db2_ref, o_ref):
        x = x_ref[...]
        h = _silu(_dot(x, w1a_ref[...]) + _dot(agg_ref[...], w1b_ref[...])
                  + b1_ref[...])
        y = _ln(_dot(h, w2_ref[...]) + b2_ref[...], g_ref[...], be_ref[...])
        g2 = y + x
        hd = _silu(_dot(g2, dw1_ref[...]) + db1_ref[...])
        o_ref[...] = _dot(hd, dw2_ref[...]) + db2_ref[...]

    return pl.pallas_call(
        body,
        grid=(NG // BN,),
        in_specs=[
            pl.BlockSpec((BN, H), lambda i: (i, 0)),
            pl.BlockSpec((BN, H), lambda i: (i, 0)),
            _full((H, H)), _full((H, H)), _full((1, H)), _full((H, H)),
            _full((1, H)), _full((1, H)), _full((1, H)),
            _full((H, H)), _full((1, H)), _full((H, OUT_GRID)),
            _full((1, OUT_GRID)),
        ],
        out_specs=[pl.BlockSpec((BN, OUT_GRID), lambda i: (i, 0))],
        out_shape=[jax.ShapeDtypeStruct((NG, OUT_GRID), jnp.float32)],
    )(g1, agg, w1a, w1b, b1, w2, b2, g, be, dw1, db1, dw2, db2)[0]


_SC_MESH = functools.partial(
    plsc.VectorSubcoreMesh, core_axis_name="c", subcore_axis_name="s")


def _sc_gather_add(a_tab, b_tab, src, dst, chunk):
    E = src.shape[0]
    epw = E // 32
    n_chunks = epw // chunk

    @functools.partial(
        pl.kernel,
        out_type=jax.ShapeDtypeStruct((E, H), jnp.float32),
        mesh=_SC_MESH(),
        scratch_types=[
            pltpu.VMEM((chunk,), jnp.int32),
            pltpu.VMEM((chunk,), jnp.int32),
            pltpu.VMEM((chunk, H), jnp.float32),
            pltpu.VMEM((chunk, H), jnp.float32),
            pltpu.SemaphoreType.DMA,
        ],
    )
    def k(a_hbm, b_hbm, s_hbm, d_hbm, o_hbm, si, di, ra, rb, sem):
        wid = lax.axis_index("s") * 2 + lax.axis_index("c")
        base = wid * epw

        @pl.loop(0, n_chunks)
        def _(t):
            off = base + t * chunk
            pltpu.sync_copy(s_hbm.at[pl.ds(off, chunk)], si)
            pltpu.sync_copy(d_hbm.at[pl.ds(off, chunk)], di)
            ca = pltpu.async_copy(a_hbm.at[si], ra, sem)
            cb = pltpu.async_copy(b_hbm.at[di], rb, sem)
            ca.wait()
            cb.wait()

            @pl.loop(0, chunk)
            def _(r):
                for j in range(H // 16):
                    sl = (r, pl.ds(j * 16, 16))
                    ra[sl] = ra[sl] + rb[sl]

            pltpu.sync_copy(ra, o_hbm.at[pl.ds(off, chunk)])

    return k(a_tab, b_tab, src, dst)


def _sc_scatter_mesh(ef, idx, zeros, chunk):
    E = ef.shape[0]
    eh = E // 2
    ept = eh // 16
    n_chunks = ept // chunk
    zr = R_MESH // 16

    @functools.partial(
        pl.kernel,
        out_type=jax.ShapeDtypeStruct((2 * R_MESH, H), jnp.float32),
        mesh=_SC_MESH(),
        scratch_types=[
            pltpu.VMEM((chunk,), jnp.int32),
            pltpu.VMEM((chunk, H), jnp.float32),
            pltpu.VMEM_SHARED((R_MESH, H), jnp.float32),
        ],
    )
    def k(ef_hbm, i_hbm, z_hbm, o_hbm, iv, rv, acc):
        c = lax.axis_index("c")
        lt = lax.axis_index("s")
        pltpu.sync_copy(z_hbm.at[pl.ds(0, zr)], acc.at[pl.ds(lt * zr, zr)])
        plsc.subcore_barrier()
        base = c * eh + lt * ept

        @pl.loop(0, n_chunks)
        def _(t):
            off = base + t * chunk
            pltpu.sync_copy(i_hbm.at[pl.ds(off, chunk)], iv)
            pltpu.sync_copy(ef_hbm.at[pl.ds(off, chunk)], rv)
            pltpu.sync_copy(rv, acc.at[iv], add=True)

        plsc.subcore_barrier()
        pltpu.sync_copy(acc.at[pl.ds(lt * zr, zr)],
                        o_hbm.at[pl.ds(c * R_MESH + lt * zr, zr)])

    return k(ef, idx, zeros)


def _sc_scatter_grid(ef, idx2, zeros, chunk):
    E = ef.shape[0]
    ept = E // 16
    n_chunks = ept // chunk

    @functools.partial(
        pl.kernel,
        out_type=jax.ShapeDtypeStruct((NG, H), jnp.float32),
        mesh=_SC_MESH(),
        scratch_types=[
            pltpu.VMEM((chunk,), jnp.int32),
            pltpu.VMEM((chunk, H), jnp.float32),
            pltpu.VMEM_SHARED((R_GRID, H), jnp.float32),
        ],
    )
    def k(ef_hbm, i_hbm, z_hbm, o_hbm, iv, rv, acc):
        c = lax.axis_index("c")
        lt = lax.axis_index("s")
        zstart = jnp.minimum(lt * 512, R_GRID - 512)
        pltpu.sync_copy(z_hbm.at[pl.ds(0, 512)], acc.at[pl.ds(zstart, 512)])
        plsc.subcore_barrier()
        base = lt * ept

        @pl.loop(0, n_chunks)
        def _(t):
            off = base + t * chunk
            pltpu.sync_copy(i_hbm.at[pl.ds(c * E + off, chunk)], iv)
            pltpu.sync_copy(ef_hbm.at[pl.ds(off, chunk)], rv)
            pltpu.sync_copy(rv, acc.at[iv], add=True)

        plsc.subcore_barrier()
        dstart = jnp.minimum(lt * 512, RD - 512)
        pltpu.sync_copy(acc.at[pl.ds(dstart, 512)],
                        o_hbm.at[pl.ds(c * RD + dstart, 512)])

    return k(ef, idx2, zeros)


def _pad_rows(x, n):
    return jnp.pad(x, ((0, n - x.shape[0]), (0, 0)))


def _pad_idx(x, n, fill):
    return jnp.pad(x.astype(jnp.int32), (0, n - x.shape[0]),
                   constant_values=fill)


def _mlp_args(p):
    r = lambda v: v.reshape(1, -1)
    return (p["w1"], r(p["b1"]), p["w2"], r(p["b2"]),
            r(p["g"]), r(p["be"]))


def kernel(grid_nfeat, mesh_nfeat, g2m_efeat, mesh_efeat, m2g_efeat, params,
           g2m_src, g2m_dst, mesh_src, mesh_dst, m2g_src, m2g_dst):
    f32 = jnp.float32
    zeros = jnp.zeros((512, H), f32)

    xt = jnp.pad(grid_nfeat[0].reshape(IN_GRID, NGD).astype(f32),
                 ((0, 0), (0, NG - NGD)))
    mn = _pad_rows(mesh_nfeat.astype(f32), NM)
    eg_in = _pad_rows(g2m_efeat.astype(f32), EG)
    em_in = _pad_rows(mesh_efeat.astype(f32), EM)
    ep_in = _pad_rows(m2g_efeat.astype(f32), EP)

    g2m_src_p = _pad_idx(g2m_src, EG, 0)
    mesh_src_p = _pad_idx(mesh_src, EM, 0)
    m2g_src_p = _pad_idx(m2g_src, EP, 0)
    g2m_dst_p = _pad_idx(g2m_dst, EG, 0)
    mesh_dst_p = _pad_idx(mesh_dst, EM, 0)
    m2g_dst_p = _pad_idx(m2g_dst, EP, 0)

    g2m_dst_sc = _pad_idx(g2m_dst, EG, NMD)
    mesh_dst_sc = _pad_idx(mesh_dst, EM, NMD)
    m2g_dst_f = _pad_idx(m2g_dst, EP, NGD)
    idx2 = jnp.concatenate([
        jnp.where((m2g_dst_f >= c * RD) & (m2g_dst_f < (c + 1) * RD),
                  m2g_dst_f - c * RD, RD)
        for c in (0, 1)
    ])

    pp = params

    w1s = pp["g2m_edge_mlp"]["w1"]
    g0, a_g2m = _grid_encoder(xt, *_mlp_args(pp["grid_enc"]), w1s[H:2 * H])
    m0, b_g2m = _mesh_encoder(mn, *_mlp_args(pp["mesh_enc"]), w1s[2 * H:])
    e_g2m = _edge_encoder(eg_in, *_mlp_args(pp["g2m_edge_enc"]))
    e_mesh = _edge_encoder(em_in, *_mlp_args(pp["mesh_edge_enc"]))
    e_m2g = _edge_encoder(ep_in, *_mlp_args(pp["m2g_edge_enc"]))

    pre0 = _sc_gather_add(a_g2m, b_g2m, g2m_src_p, g2m_dst_p, 104)
    pe = pp["g2m_edge_mlp"]
    ef = _edge_mlp(e_g2m, pre0, pe["w1"][:H], *_mlp_args(pe)[1:])
    parts = _sc_scatter_mesh(ef, g2m_dst_sc, zeros, 104)
    pn = pp["g2m_node_mlp"]
    w1e0 = pp["proc"][0]["edge"]["w1"]
    m1, a_p, b_p = _node_mlp_mesh(m0, parts, pn["w1"][:H], pn["w1"][H:],
                                  *_mlp_args(pn)[1:], w1e0[H:2 * H],
                                  w1e0[2 * H:])
    pg = pp["g2m_grid_mlp"]
    g1, b_m2g = _grid_mlp(g0, *_mlp_args(pg), pp["m2g_edge_mlp"]["w1"][2 * H:])

    m = m1
    for i in range(N_PROC):
        pre0 = _sc_gather_add(a_p, b_p, mesh_src_p, mesh_dst_p, 128)
        pe = pp["proc"][i]["edge"]
        ef = _edge_mlp(e_mesh, pre0, pe["w1"][:H], *_mlp_args(pe)[1:])
        e_mesh = ef
        parts = _sc_scatter_mesh(ef, mesh_dst_sc, zeros, 128)
        pn = pp["proc"][i]["node"]
        if i + 1 < N_PROC:
            w1n = pp["proc"][i + 1]["edge"]["w1"]
            wa, wb = w1n[H:2 * H], w1n[2 * H:]
        else:
            w1n = pp["m2g_edge_mlp"]["w1"]
            wa, wb = w1n[H:2 * H], w1n[H:2 * H]
        m, a_p, b_p = _node_mlp_mesh(m, parts, pn["w1"][:H], pn["w1"][H:],
                                     *_mlp_args(pn)[1:], wa, wb)

    pre0 = _sc_gather_add(a_p, b_m2g, m2g_src_p, m2g_dst_p, 128)
    pe = pp["m2g_edge_mlp"]
    ef = _edge_mlp(e_m2g, pre0, pe["w1"][:H], *_mlp_args(pe)[1:])
    agg = _sc_scatter_grid(ef, idx2, zeros, 128)
    pn = pp["m2g_node_mlp"]
    pd = pp["decoder"]
    out = _grid_node_decoder(g1, agg, pn["w1"][:H], pn["w1"][H:],
                             *_mlp_args(pn)[1:],
                             pd["w1"], pd["b1"].reshape(1, -1),
                             pd["w2"], pd["b2"].reshape(1, -1))

    return out[:NGD].T.reshape(1, OUT_GRID, RES_H, RES_W)


# speedup vs baseline: 1.0592x; 1.0592x over previous
import functools

import jax
import jax.numpy as jnp
from jax import lax
from jax.experimental import pallas as pl
from jax.experimental.pallas import tpu as pltpu
from jax.experimental.pallas import tpu_sc as plsc

H = 256
IN_GRID = 474
OUT_GRID = 227
RES_H, RES_W = 91, 180
NGD = RES_H * RES_W
NG = 16384
NMD = 2562
NM = 2568
EG = 26624
EM = 20480
EP = 49152
N_PROC = 4

R_MESH = NM
RD = 8190
GB1 = 8192
GQ = 4096

BN = 512

_DOT = functools.partial(
    lax.dot_general,
    precision=lax.Precision.HIGHEST,
    preferred_element_type=jnp.float32,
)


def _dot(a, b):
    return _DOT(a, b, dimension_numbers=(((1,), (0,)), ((), ())))


def _dot_t(a, b):
    return _DOT(a, b, dimension_numbers=(((0,), (0,)), ((), ())))


def _ln(y, g, b):
    mu = jnp.mean(y, axis=-1, keepdims=True)
    var = jnp.mean((y - mu) ** 2, axis=-1, keepdims=True)
    return (y - mu) * lax.rsqrt(var + 1e-5) * g + b


def _silu(x):
    return x * jax.nn.sigmoid(x)


def _full(shape):
    nd = len(shape)
    return pl.BlockSpec(shape, lambda i: (0,) * nd)


def _grid_encoder(xt, w1, b1, w2, b2, g, be, wa):
    def body(x_ref, w1_ref, b1_ref, w2_ref, b2_ref, g_ref, be_ref, wa_ref,
             g0_ref, a_ref):
        h = _silu(_dot_t(x_ref[...], w1_ref[...]) + b1_ref[...])
        y = _ln(_dot(h, w2_ref[...]) + b2_ref[...], g_ref[...], be_ref[...])
        g0_ref[...] = y
        a_ref[...] = _dot(y, wa_ref[...])

    return pl.pallas_call(
        body,
        grid=(NG // BN,),
        in_specs=[
            pl.BlockSpec((IN_GRID, BN), lambda i: (0, i)),
            _full((IN_GRID, H)), _full((1, H)), _full((H, H)), _full((1, H)),
            _full((1, H)), _full((1, H)), _full((H, H)),
        ],
        out_specs=[
            pl.BlockSpec((BN, H), lambda i: (i, 0)),
            pl.BlockSpec((BN, H), lambda i: (i, 0)),
        ],
        out_shape=[
            jax.ShapeDtypeStruct((NG, H), jnp.float32),
            jax.ShapeDtypeStruct((NG, H), jnp.float32),
        ],
    )(xt, w1, b1, w2, b2, g, be, wa)


def _mesh_encoder(x, w1, b1, w2, b2, g, be, wb):
    def body(x_ref, w1_ref, b1_ref, w2_ref, b2_ref, g_ref, be_ref, wb_ref,
             m_ref, b_out_ref):
        h = _silu(_dot(x_ref[...], w1_ref[...]) + b1_ref[...])
        y = _ln(_dot(h, w2_ref[...]) + b2_ref[...], g_ref[...], be_ref[...])
        m_ref[...] = y
        b_out_ref[...] = _dot(y, wb_ref[...])

    return pl.pallas_call(
        body,
        out_shape=[
            jax.ShapeDtypeStruct((NM, H), jnp.float32),
            jax.ShapeDtypeStruct((NM, H), jnp.float32),
        ],
    )(x, w1, b1, w2, b2, g, be, wb)


def _edge_encoder(x, w1, b1, w2, b2, g, be):
    E = x.shape[0]

    def body(x_ref, w1_ref, b1_ref, w2_ref, b2_ref, g_ref, be_ref, e_ref):
        h = _silu(_dot(x_ref[...], w1_ref[...]) + b1_ref[...])
        e_ref[...] = _ln(_dot(h, w2_ref[...]) + b2_ref[...],
                         g_ref[...], be_ref[...])

    return pl.pallas_call(
        body,
        grid=(E // BN,),
        in_specs=[
            pl.BlockSpec((BN, 4), lambda i: (i, 0)),
            _full((4, H)), _full((1, H)), _full((H, H)), _full((1, H)),
            _full((1, H)), _full((1, H)),
        ],
        out_specs=[pl.BlockSpec((BN, H), lambda i: (i, 0))],
        out_shape=[jax.ShapeDtypeStruct((E, H), jnp.float32)],
    )(x, w1, b1, w2, b2, g, be)[0]


def _edge_mlp(e, pre0, w1a, b1, w2, b2, g, be):
    E = e.shape[0]

    def body(e_ref, p_ref, w1_ref, b1_ref, w2_ref, b2_ref, g_ref, be_ref,
             ef_ref):
        x = e_ref[...]
        h = _silu(_dot(x, w1_ref[...]) + p_ref[...] + b1_ref[...])
        y = _ln(_dot(h, w2_ref[...]) + b2_ref[...], g_ref[...], be_ref[...])
        ef_ref[...] = y + x

    return pl.pallas_call(
        body,
        grid=(E // BN,),
        in_specs=[
            pl.BlockSpec((BN, H), lambda i: (i, 0)),
            pl.BlockSpec((BN, H), lambda i: (i, 0)),
            _full((H, H)), _full((1, H)), _full((H, H)), _full((1, H)),
            _full((1, H)), _full((1, H)),
        ],
        out_specs=[pl.BlockSpec((BN, H), lambda i: (i, 0))],
        out_shape=[jax.ShapeDtypeStruct((E, H), jnp.float32)],
    )(e, pre0, w1a, b1, w2, b2, g, be)[0]


def _node_mlp_mesh(m, parts, w1a, w1b, b1, w2, b2, g, be, wa, wb):
    def body(m_ref, p_ref, w1a_ref, w1b_ref, b1_ref, w2_ref, b2_ref, g_ref,
             be_ref, wa_ref, wb_ref, m2_ref, a_ref, b_ref):
        x = m_ref[...]
        agg = p_ref[0:NM, :] + p_ref[R_MESH:R_MESH + NM, :]
        h = _silu(_dot(x, w1a_ref[...]) + _dot(agg, w1b_ref[...])
                  + b1_ref[...])
        y = _ln(_dot(h, w2_ref[...]) + b2_ref[...], g_ref[...], be_ref[...])
        m2 = y + x
        m2_ref[...] = m2
        a_ref[...] = _dot(m2, wa_ref[...])
        b_ref[...] = _dot(m2, wb_ref[...])

    return pl.pallas_call(
        body,
        out_shape=[
            jax.ShapeDtypeStruct((NM, H), jnp.float32),
            jax.ShapeDtypeStruct((NM, H), jnp.float32),
            jax.ShapeDtypeStruct((NM, H), jnp.float32),
        ],
    )(m, parts, w1a, w1b, b1, w2, b2, g, be, wa, wb)


def _grid_mlp(g0, w1, b1, w2, b2, g, be, wb):
    def body(x_ref, w1_ref, b1_ref, w2_ref, b2_ref, g_ref, be_ref, wb_ref,
             g1_ref, b_ref):
        x = x_ref[...]
        h = _silu(_dot(x, w1_ref[...]) + b1_ref[...])
        y = _ln(_dot(h, w2_ref[...]) + b2_ref[...], g_ref[...], be_ref[...])
        g1 = y + x
        g1_ref[...] = g1
        b_ref[...] = _dot(g1, wb_ref[...])

    return pl.pallas_call(
        body,
        grid=(NG // BN,),
        in_specs=[
            pl.BlockSpec((BN, H), lambda i: (i, 0)),
            _full((H, H)), _full((1, H)), _full((H, H)), _full((1, H)),
            _full((1, H)), _full((1, H)), _full((H, H)),
        ],
        out_specs=[
            pl.BlockSpec((BN, H), lambda i: (i, 0)),
            pl.BlockSpec((BN, H), lambda i: (i, 0)),
        ],
        out_shape=[
            jax.ShapeDtypeStruct((NG, H), jnp.float32),
            jax.ShapeDtypeStruct((NG, H), jnp.float32),
        ],
    )(g0, w1, b1, w2, b2, g, be, wb)


def _grid_node_decoder(g1, agg, w1a, w1b, b1, w2, b2, g, be,
                       dw1, db1, dw2, db2):
    def body(x_ref, agg_ref, w1a_ref, w1b_ref, b1_ref, w2_ref, b2_ref,
             g_ref, be_ref, dw1_ref, db1_ref, dw2_ref, db2_ref, o_ref):
        x = x_ref[...]
        h = _silu(_dot(x, w1a_ref[...]) + _dot(agg_ref[...], w1b_ref[...])
                  + b1_ref[...])
        y = _ln(_dot(h, w2_ref[...]) + b2_ref[...], g_ref[...], be_ref[...])
        g2 = y + x
        hd = _silu(_dot(g2, dw1_ref[...]) + db1_ref[...])
        o_ref[...] = _dot(hd, dw2_ref[...]) + db2_ref[...]

    return pl.pallas_call(
        body,
        grid=(NG // BN,),
        in_specs=[
            pl.BlockSpec((BN, H), lambda i: (i, 0)),
            pl.BlockSpec((BN, H), lambda i: (i, 0)),
            _full((H, H)), _full((H, H)), _full((1, H)), _full((H, H)),
            _full((1, H)), _full((1, H)), _full((1, H)),
            _full((H, H)), _full((1, H)), _full((H, OUT_GRID)),
            _full((1, OUT_GRID)),
        ],
        out_specs=[pl.BlockSpec((BN, OUT_GRID), lambda i: (i, 0))],
        out_shape=[jax.ShapeDtypeStruct((NG, OUT_GRID), jnp.float32)],
    )(g1, agg, w1a, w1b, b1, w2, b2, g, be, dw1, db1, dw2, db2)[0]


_SC_MESH = functools.partial(
    plsc.VectorSubcoreMesh, core_axis_name="c", subcore_axis_name="s")


def _sc_gather_add(a_tab, b_tab, src, dst, chunk):
    E = src.shape[0]
    epw = E // 32
    n_chunks = epw // chunk

    @functools.partial(
        pl.kernel,
        out_type=jax.ShapeDtypeStruct((E, 128), jnp.float32),
        mesh=_SC_MESH(),
        scratch_types=[
            pltpu.VMEM((chunk,), jnp.int32),
            pltpu.VMEM((chunk,), jnp.int32),
            pltpu.VMEM((chunk, 128), jnp.float32),
            pltpu.VMEM((chunk, 128), jnp.float32),
            pltpu.SemaphoreType.DMA,
        ],
    )
    def k(a_hbm, b_hbm, s_hbm, d_hbm, o_hbm, si, di, ra, rb, sem):
        wid = lax.axis_index("s") * 2 + lax.axis_index("c")
        base = wid * epw

        @pl.loop(0, n_chunks)
        def _(t):
            off = base + t * chunk
            pltpu.sync_copy(s_hbm.at[pl.ds(off, chunk)], si)
            pltpu.sync_copy(d_hbm.at[pl.ds(off, chunk)], di)
            ca = pltpu.async_copy(a_hbm.at[si], ra, sem)
            cb = pltpu.async_copy(b_hbm.at[di], rb, sem)
            ca.wait()
            cb.wait()

            @pl.loop(0, chunk)
            def _(r):
                for j in range(128 // 16):
                    sl = (r, pl.ds(j * 16, 16))
                    ra[sl] = ra[sl] + rb[sl]

            pltpu.sync_copy(ra, o_hbm.at[pl.ds(off, chunk)])

    return k(a_tab, b_tab, src, dst)


def _sc_scatter_mesh(ef, idx, zeros, chunk):
    E = ef.shape[0]
    eh = E // 2
    spw = eh // 16
    n_chunks = spw // chunk
    acc_n = 2 * R_MESH
    zb, nz = 856, 6

    @functools.partial(
        pl.kernel,
        out_type=jax.ShapeDtypeStruct((2 * acc_n, 128), jnp.float32),
        mesh=_SC_MESH(),
        scratch_types=[
            pltpu.VMEM((chunk,), jnp.int32),
            pltpu.VMEM((chunk, 128), jnp.float32),
            pltpu.VMEM_SHARED((acc_n, 128), jnp.float32),
        ],
    )
    def k(ef_hbm, i_hbm, z_hbm, o_hbm, iv, rv, acc):
        c = lax.axis_index("c")
        lt = lax.axis_index("s")

        @pl.when(lt < nz)
        def _():
            pltpu.sync_copy(z_hbm.at[pl.ds(0, zb)],
                            acc.at[pl.ds(lt * zb, zb)])
        plsc.subcore_barrier()

        @pl.loop(0, n_chunks)
        def _(t):
            off = c * eh + lt * spw + t * chunk
            pltpu.sync_copy(i_hbm.at[pl.ds(off, chunk)], iv)
            pltpu.sync_copy(ef_hbm.at[pl.ds(off, chunk)], rv)
            pltpu.sync_copy(rv, acc.at[iv], add=True)

        plsc.subcore_barrier()

        @pl.when(lt < nz)
        def _():
            pltpu.sync_copy(acc.at[pl.ds(lt * zb, zb)],
                            o_hbm.at[pl.ds(c * acc_n + lt * zb, zb)])

    return k(ef, idx, zeros)


def _sc_scatter_grid(ef, idx2, zeros, chunk):
    E = ef.shape[0]
    spw = E // 16
    n_chunks = spw // chunk
    acc_r = 2 * GQ
    acc_n = acc_r + 8
    zb = acc_r // 16

    @functools.partial(
        pl.kernel,
        out_type=jax.ShapeDtypeStruct((2 * acc_r, 128), jnp.float32),
        mesh=_SC_MESH(),
        scratch_types=[
            pltpu.VMEM((chunk,), jnp.int32),
            pltpu.VMEM((chunk, 128), jnp.float32),
            pltpu.VMEM_SHARED((acc_n, 128), jnp.float32),
        ],
    )
    def k(ef_hbm, i_hbm, z_hbm, o_hbm, iv, rv, acc):
        c = lax.axis_index("c")
        lt = lax.axis_index("s")
        pltpu.sync_copy(z_hbm.at[pl.ds(0, zb)], acc.at[pl.ds(lt * zb, zb)])
        plsc.subcore_barrier()

        @pl.loop(0, n_chunks)
        def _(t):
            off = lt * spw + t * chunk
            pltpu.sync_copy(i_hbm.at[pl.ds(c * E + off, chunk)], iv)
            pltpu.sync_copy(ef_hbm.at[pl.ds(off, chunk)], rv)
            pltpu.sync_copy(rv, acc.at[iv], add=True)

        plsc.subcore_barrier()
        pltpu.sync_copy(acc.at[pl.ds(lt * zb, zb)],
                        o_hbm.at[pl.ds(c * acc_r + lt * zb, zb)])

    return k(ef, idx2, zeros)


def _pad_rows(x, n):
    return jnp.pad(x, ((0, n - x.shape[0]), (0, 0)))


def _pad_idx(x, n, fill):
    return jnp.pad(x.astype(jnp.int32), (0, n - x.shape[0]),
                   constant_values=fill)


def _mlp_args(p):
    r = lambda v: v.reshape(1, -1)
    return (p["w1"], r(p["b1"]), p["w2"], r(p["b2"]),
            r(p["g"]), r(p["be"]))


def kernel(grid_nfeat, mesh_nfeat, g2m_efeat, mesh_efeat, m2g_efeat, params,
           g2m_src, g2m_dst, mesh_src, mesh_dst, m2g_src, m2g_dst):
    f32 = jnp.float32
    zeros = jnp.zeros((1024, 128), f32)

    xr = grid_nfeat[0].reshape(IN_GRID, NGD).astype(f32)
    zc = jnp.zeros((IN_GRID, 2), f32)
    xt = jnp.concatenate([xr[:, :RD], zc, xr[:, RD:], zc], axis=1)
    mn = _pad_rows(mesh_nfeat.astype(f32), NM)
    eg_in = _pad_rows(g2m_efeat.astype(f32), EG)
    em_in = _pad_rows(mesh_efeat.astype(f32), EM)
    ep_in = _pad_rows(m2g_efeat.astype(f32), EP)

    def to_layout(i):
        i = i.astype(jnp.int32)
        return jnp.where(i < RD, i, i + 2)

    g2m_src_p = _pad_idx(to_layout(g2m_src), EG, 0)
    mesh_src_p = _pad_idx(mesh_src, EM, 0)
    m2g_src_p = _pad_idx(m2g_src, EP, 0)
    g2m_dst_p = _pad_idx(g2m_dst, EG, 0)
    mesh_dst_p = _pad_idx(mesh_dst, EM, 0)
    m2g_dst_p = _pad_idx(to_layout(m2g_dst), EP, 0)

    def dbl(i):
        return jnp.stack([2 * i, 2 * i + 1], axis=-1).reshape(-1)

    g2m_dst_sc = dbl(_pad_idx(g2m_dst, EG, NMD))
    mesh_dst_sc = dbl(_pad_idx(mesh_dst, EM, NMD))
    mesh_src2 = dbl(mesh_src_p)
    mesh_dst2 = dbl(mesh_dst_p)
    gl = _pad_idx(to_layout(m2g_dst), EP, GB1 + RD)

    def grid_idx(q):
        secs = []
        for r in (2 * q, 2 * q + 1):
            base = r * GQ
            secs.append(jnp.where((gl >= base) & (gl < base + GQ),
                                  gl - base, GQ))
        return dbl(jnp.concatenate(secs))

    pp = params

    w1s = pp["g2m_edge_mlp"]["w1"]
    g0, a_g2m = _grid_encoder(xt, *_mlp_args(pp["grid_enc"]), w1s[H:2 * H])
    m0, b_g2m = _mesh_encoder(mn, *_mlp_args(pp["mesh_enc"]), w1s[2 * H:])
    e_g2m = _edge_encoder(eg_in, *_mlp_args(pp["g2m_edge_enc"]))
    e_mesh = _edge_encoder(em_in, *_mlp_args(pp["mesh_edge_enc"]))
    e_m2g = _edge_encoder(ep_in, *_mlp_args(pp["m2g_edge_enc"]))

    pre0 = _sc_gather_add(a_g2m.reshape(2 * NG, 128),
                          b_g2m.reshape(2 * NM, 128),
                          dbl(g2m_src_p), dbl(g2m_dst_p), 208).reshape(EG, H)
    pe = pp["g2m_edge_mlp"]
    ef = _edge_mlp(e_g2m, pre0, pe["w1"][:H], *_mlp_args(pe)[1:])
    parts = _sc_scatter_mesh(ef.reshape(2 * EG, 128), g2m_dst_sc,
                             zeros, 104).reshape(2 * R_MESH, H)
    pn = pp["g2m_node_mlp"]
    w1e0 = pp["proc"][0]["edge"]["w1"]
    m1, a_p, b_p = _node_mlp_mesh(m0, parts, pn["w1"][:H], pn["w1"][H:],
                                  *_mlp_args(pn)[1:], w1e0[H:2 * H],
                                  w1e0[2 * H:])
    pg = pp["g2m_grid_mlp"]
    g1, b_m2g = _grid_mlp(g0, *_mlp_args(pg), pp["m2g_edge_mlp"]["w1"][2 * H:])

    m = m1
    for i in range(N_PROC):
        pre0 = _sc_gather_add(a_p.reshape(2 * NM, 128),
                              b_p.reshape(2 * NM, 128),
                              mesh_src2, mesh_dst2, 256).reshape(EM, H)
        pe = pp["proc"][i]["edge"]
        ef = _edge_mlp(e_mesh, pre0, pe["w1"][:H], *_mlp_args(pe)[1:])
        e_mesh = ef
        parts = _sc_scatter_mesh(ef.reshape(2 * EM, 128), mesh_dst_sc,
                                 zeros, 128).reshape(2 * R_MESH, H)
        pn = pp["proc"][i]["node"]
        if i + 1 < N_PROC:
            w1n = pp["proc"][i + 1]["edge"]["w1"]
            wa, wb = w1n[H:2 * H], w1n[2 * H:]
        else:
            w1n = pp["m2g_edge_mlp"]["w1"]
            wa, wb = w1n[H:2 * H], w1n[H:2 * H]
        m, a_p, b_p = _node_mlp_mesh(m, parts, pn["w1"][:H], pn["w1"][H:],
                                     *_mlp_args(pn)[1:], wa, wb)

    pre0 = _sc_gather_add(a_p.reshape(2 * NM, 128),
                          b_m2g.reshape(2 * NG, 128),
                          dbl(m2g_src_p), dbl(m2g_dst_p), 256).reshape(EP, H)
    pe = pp["m2g_edge_mlp"]
    ef = _edge_mlp(e_m2g, pre0, pe["w1"][:H], *_mlp_args(pe)[1:])
    ef2 = ef.reshape(2 * EP, 128)
    agg = jnp.concatenate([
        _sc_scatter_grid(ef2, grid_idx(0), zeros, 128),
        _sc_scatter_grid(ef2, grid_idx(1), zeros, 128),
    ]).reshape(NG, H)
    pn = pp["m2g_node_mlp"]
    pd = pp["decoder"]
    out = _grid_node_decoder(g1, agg, pn["w1"][:H], pn["w1"][H:],
                             *_mlp_args(pn)[1:],
                             pd["w1"], pd["b1"].reshape(1, -1),
                             pd["w2"], pd["b2"].reshape(1, -1))

    out = jnp.concatenate([out[:RD], out[GB1:GB1 + RD]])
    return out.T.reshape(1, OUT_GRID, RES_H, RES_W)


# speedup vs baseline: 1.0843x; 1.0238x over previous
import functools

import jax
import jax.numpy as jnp
from jax import lax
from jax.experimental import pallas as pl
from jax.experimental.pallas import tpu as pltpu
from jax.experimental.pallas import tpu_sc as plsc

H = 256
IN_GRID = 474
OUT_GRID = 227
RES_H, RES_W = 91, 180
NGD = RES_H * RES_W
NG = 16384
NMD = 2562
NM = 2568
EG = 26624
EM = 20480
EP = 49152
N_PROC = 4

R_MESH = NM
RD = 8190
GB1 = 8192
GQ = 4096

BN = 512

_DOT = functools.partial(
    lax.dot_general,
    precision=lax.Precision.HIGHEST,
    preferred_element_type=jnp.float32,
)


def _dot(a, b):
    return _DOT(a, b, dimension_numbers=(((1,), (0,)), ((), ())))


def _dot_t(a, b):
    return _DOT(a, b, dimension_numbers=(((0,), (0,)), ((), ())))


def _ln(y, g, b):
    mu = jnp.mean(y, axis=-1, keepdims=True)
    var = jnp.mean((y - mu) ** 2, axis=-1, keepdims=True)
    return (y - mu) * lax.rsqrt(var + 1e-5) * g + b


def _silu(x):
    return x * jax.nn.sigmoid(x)


def _full(shape):
    nd = len(shape)
    return pl.BlockSpec(shape, lambda i: (0,) * nd)


def _grid_encoder(xt, w1, b1, w2, b2, g, be, wa):
    def body(x_ref, w1_ref, b1_ref, w2_ref, b2_ref, g_ref, be_ref, wa_ref,
             g0_ref, a_ref):
        h = _silu(_dot_t(x_ref[...], w1_ref[...]) + b1_ref[...])
        y = _ln(_dot(h, w2_ref[...]) + b2_ref[...], g_ref[...], be_ref[...])
        g0_ref[...] = y
        a_ref[...] = _dot(y, wa_ref[...])

    return pl.pallas_call(
        body,
        grid=(NG // BN,),
        in_specs=[
            pl.BlockSpec((IN_GRID, BN), lambda i: (0, i)),
            _full((IN_GRID, H)), _full((1, H)), _full((H, H)), _full((1, H)),
            _full((1, H)), _full((1, H)), _full((H, H)),
        ],
        out_specs=[
            pl.BlockSpec((BN, H), lambda i: (i, 0)),
            pl.BlockSpec((BN, H), lambda i: (i, 0)),
        ],
        out_shape=[
            jax.ShapeDtypeStruct((NG, H), jnp.float32),
            jax.ShapeDtypeStruct((NG, H), jnp.float32),
        ],
    )(xt, w1, b1, w2, b2, g, be, wa)


def _mesh_encoder(x, w1, b1, w2, b2, g, be, wb):
    def body(x_ref, w1_ref, b1_ref, w2_ref, b2_ref, g_ref, be_ref, wb_ref,
             m_ref, b_out_ref):
        h = _silu(_dot(x_ref[...], w1_ref[...]) + b1_ref[...])
        y = _ln(_dot(h, w2_ref[...]) + b2_ref[...], g_ref[...], be_ref[...])
        m_ref[...] = y
        b_out_ref[...] = _dot(y, wb_ref[...])

    return pl.pallas_call(
        body,
        out_shape=[
            jax.ShapeDtypeStruct((NM, H), jnp.float32),
            jax.ShapeDtypeStruct((NM, H), jnp.float32),
        ],
    )(x, w1, b1, w2, b2, g, be, wb)


def _edge_encoder(x, w1, b1, w2, b2, g, be):
    E = x.shape[0]

    def body(x_ref, w1_ref, b1_ref, w2_ref, b2_ref, g_ref, be_ref, e_ref):
        h = _silu(_dot(x_ref[...], w1_ref[...]) + b1_ref[...])
        e_ref[...] = _ln(_dot(h, w2_ref[...]) + b2_ref[...],
                         g_ref[...], be_ref[...])

    return pl.pallas_call(
        body,
        grid=(E // BN,),
        in_specs=[
            pl.BlockSpec((BN, 4), lambda i: (i, 0)),
            _full((4, H)), _full((1, H)), _full((H, H)), _full((1, H)),
            _full((1, H)), _full((1, H)),
        ],
        out_specs=[pl.BlockSpec((BN, H), lambda i: (i, 0))],
        out_shape=[jax.ShapeDtypeStruct((E, H), jnp.float32)],
    )(x, w1, b1, w2, b2, g, be)[0]


def _edge_mlp(e, pre0, w1a, b1, w2, b2, g, be):
    E = e.shape[0]

    def body(e_ref, p_ref, w1_ref, b1_ref, w2_ref, b2_ref, g_ref, be_ref,
             ef_ref):
        x = e_ref[...]
        h = _silu(_dot(x, w1_ref[...]) + p_ref[...] + b1_ref[...])
        y = _ln(_dot(h, w2_ref[...]) + b2_ref[...], g_ref[...], be_ref[...])
        ef_ref[...] = y + x

    return pl.pallas_call(
        body,
        grid=(E // BN,),
        in_specs=[
            pl.BlockSpec((BN, H), lambda i: (i, 0)),
            pl.BlockSpec((BN, H), lambda i: (i, 0)),
            _full((H, H)), _full((1, H)), _full((H, H)), _full((1, H)),
            _full((1, H)), _full((1, H)),
        ],
        out_specs=[pl.BlockSpec((BN, H), lambda i: (i, 0))],
        out_shape=[jax.ShapeDtypeStruct((E, H), jnp.float32)],
    )(e, pre0, w1a, b1, w2, b2, g, be)[0]


def _node_mlp_mesh(m, parts, w1a, w1b, b1, w2, b2, g, be, wa, wb):
    def body(m_ref, p_ref, w1a_ref, w1b_ref, b1_ref, w2_ref, b2_ref, g_ref,
             be_ref, wa_ref, wb_ref, m2_ref, a_ref, b_ref):
        x = m_ref[...]
        agg = p_ref[0:NM, :] + p_ref[R_MESH:R_MESH + NM, :]
        h = _silu(_dot(x, w1a_ref[...]) + _dot(agg, w1b_ref[...])
                  + b1_ref[...])
        y = _ln(_dot(h, w2_ref[...]) + b2_ref[...], g_ref[...], be_ref[...])
        m2 = y + x
        m2_ref[...] = m2
        a_ref[...] = _dot(m2, wa_ref[...])
        b_ref[...] = _dot(m2, wb_ref[...])

    return pl.pallas_call(
        body,
        out_shape=[
            jax.ShapeDtypeStruct((NM, H), jnp.float32),
            jax.ShapeDtypeStruct((NM, H), jnp.float32),
            jax.ShapeDtypeStruct((NM, H), jnp.float32),
        ],
    )(m, parts, w1a, w1b, b1, w2, b2, g, be, wa, wb)


def _grid_mlp(g0, w1, b1, w2, b2, g, be, wb):
    def body(x_ref, w1_ref, b1_ref, w2_ref, b2_ref, g_ref, be_ref, wb_ref,
             g1_ref, b_ref):
        x = x_ref[...]
        h = _silu(_dot(x, w1_ref[...]) + b1_ref[...])
        y = _ln(_dot(h, w2_ref[...]) + b2_ref[...], g_ref[...], be_ref[...])
        g1 = y + x
        g1_ref[...] = g1
        b_ref[...] = _dot(g1, wb_ref[...])

    return pl.pallas_call(
        body,
        grid=(NG // BN,),
        in_specs=[
            pl.BlockSpec((BN, H), lambda i: (i, 0)),
            _full((H, H)), _full((1, H)), _full((H, H)), _full((1, H)),
            _full((1, H)), _full((1, H)), _full((H, H)),
        ],
        out_specs=[
            pl.BlockSpec((BN, H), lambda i: (i, 0)),
            pl.BlockSpec((BN, H), lambda i: (i, 0)),
        ],
        out_shape=[
            jax.ShapeDtypeStruct((NG, H), jnp.float32),
            jax.ShapeDtypeStruct((NG, H), jnp.float32),
        ],
    )(g0, w1, b1, w2, b2, g, be, wb)


def _grid_node_decoder(g1, agg, w1a, w1b, b1, w2, b2, g, be,
                       dw1, db1, dw2, db2):
    def body(x_ref, agg_ref, w1a_ref, w1b_ref, b1_ref, w2_ref, b2_ref,
             g_ref, be_ref, dw1_ref, db1_ref, dw2_ref, db2_ref, o_ref):
        x = x_ref[...]
        h = _silu(_dot(x, w1a_ref[...]) + _dot(agg_ref[...], w1b_ref[...])
                  + b1_ref[...])
        y = _ln(_dot(h, w2_ref[...]) + b2_ref[...], g_ref[...], be_ref[...])
        g2 = y + x
        hd = _silu(_dot(g2, dw1_ref[...]) + db1_ref[...])
        o_ref[...] = _dot(hd, dw2_ref[...]) + db2_ref[...]

    return pl.pallas_call(
        body,
        grid=(NG // BN,),
        in_specs=[
            pl.BlockSpec((BN, H), lambda i: (i, 0)),
            pl.BlockSpec((BN, H), lambda i: (i, 0)),
            _full((H, H)), _full((H, H)), _full((1, H)), _full((H, H)),
            _full((1, H)), _full((1, H)), _full((1, H)),
            _full((H, H)), _full((1, H)), _full((H, OUT_GRID)),
            _full((1, OUT_GRID)),
        ],
        out_specs=[pl.BlockSpec((BN, OUT_GRID), lambda i: (i, 0))],
        out_shape=[jax.ShapeDtypeStruct((NG, OUT_GRID), jnp.float32)],
    )(g1, agg, w1a, w1b, b1, w2, b2, g, be, dw1, db1, dw2, db2)[0]


_SC_MESH = functools.partial(
    plsc.VectorSubcoreMesh, core_axis_name="c", subcore_axis_name="s")


def _sc_gather_add(a_tab, b_tab, src, dst, chunk):
    E = src.shape[0]
    epw = E // 32
    n_chunks = epw // chunk

    @functools.partial(
        pl.kernel,
        out_type=jax.ShapeDtypeStruct((E, 128), jnp.float32),
        mesh=_SC_MESH(),
        scratch_types=[
            pltpu.VMEM((chunk,), jnp.int32),
            pltpu.VMEM((chunk,), jnp.int32),
            pltpu.VMEM((chunk, 128), jnp.float32),
            pltpu.VMEM((chunk, 128), jnp.float32),
            pltpu.SemaphoreType.DMA,
        ],
    )
    def k(a_hbm, b_hbm, s_hbm, d_hbm, o_hbm, si, di, ra, rb, sem):
        wid = lax.axis_index("s") * 2 + lax.axis_index("c")
        base = wid * epw

        @pl.loop(0, n_chunks)
        def _(t):
            off = base + t * chunk
            pltpu.sync_copy(s_hbm.at[pl.ds(off, chunk)], si)
            pltpu.sync_copy(d_hbm.at[pl.ds(off, chunk)], di)
            ca = pltpu.async_copy(a_hbm.at[si], ra, sem)
            cb = pltpu.async_copy(b_hbm.at[di], rb, sem)
            ca.wait()
            cb.wait()

            @pl.loop(0, chunk)
            def _(r):
                for j in range(128 // 16):
                    sl = (r, pl.ds(j * 16, 16))
                    ra[sl] = ra[sl] + rb[sl]

            pltpu.sync_copy(ra, o_hbm.at[pl.ds(off, chunk)])

    return k(a_tab, b_tab, src, dst)


def _sc_scatter_mesh(ef, idx, zeros, chunk):
    E = ef.shape[0]
    eh = E // 2
    spw = eh // 16
    n_chunks = spw // chunk
    acc_n = 2 * R_MESH
    zb, nz = 856, 6

    @functools.partial(
        pl.kernel,
        out_type=jax.ShapeDtypeStruct((2 * acc_n, 128), jnp.float32),
        mesh=_SC_MESH(),
        scratch_types=[
            pltpu.VMEM((chunk,), jnp.int32),
            pltpu.VMEM((chunk, 128), jnp.float32),
            pltpu.VMEM_SHARED((acc_n, 128), jnp.float32),
        ],
    )
    def k(ef_hbm, i_hbm, z_hbm, o_hbm, iv, rv, acc):
        c = lax.axis_index("c")
        lt = lax.axis_index("s")

        @pl.when(lt < nz)
        def _():
            pltpu.sync_copy(z_hbm.at[pl.ds(0, zb)],
                            acc.at[pl.ds(lt * zb, zb)])
        plsc.subcore_barrier()

        @pl.loop(0, n_chunks)
        def _(t):
            off = c * eh + lt * spw + t * chunk
            pltpu.sync_copy(i_hbm.at[pl.ds(off, chunk)], iv)
            pltpu.sync_copy(ef_hbm.at[pl.ds(off, chunk)], rv)
            pltpu.sync_copy(rv, acc.at[iv], add=True)

        plsc.subcore_barrier()

        @pl.when(lt < nz)
        def _():
            pltpu.sync_copy(acc.at[pl.ds(lt * zb, zb)],
                            o_hbm.at[pl.ds(c * acc_n + lt * zb, zb)])

    return k(ef, idx, zeros)


def _sc_scatter_grid(ef, idx2, zeros, chunk):
    E = ef.shape[0]
    spw = E // 16
    n_chunks = spw // chunk
    acc_r = 2 * GQ
    acc_n = acc_r + 8
    zb = acc_r // 16

    @functools.partial(
        pl.kernel,
        out_type=jax.ShapeDtypeStruct((2 * acc_r, 128), jnp.float32),
        mesh=_SC_MESH(),
        scratch_types=[
            pltpu.VMEM((chunk,), jnp.int32),
            pltpu.VMEM((chunk, 128), jnp.float32),
            pltpu.VMEM_SHARED((acc_n, 128), jnp.float32),
        ],
    )
    def k(ef_hbm, i_hbm, z_hbm, o_hbm, iv, rv, acc):
        c = lax.axis_index("c")
        lt = lax.axis_index("s")
        pltpu.sync_copy(z_hbm.at[pl.ds(0, zb)], acc.at[pl.ds(lt * zb, zb)])
        plsc.subcore_barrier()

        @pl.loop(0, n_chunks)
        def _(t):
            off = lt * spw + t * chunk
            pltpu.sync_copy(i_hbm.at[pl.ds(c * E + off, chunk)], iv)
            pltpu.sync_copy(ef_hbm.at[pl.ds(off, chunk)], rv)
            pltpu.sync_copy(rv, acc.at[iv], add=True)

        plsc.subcore_barrier()
        pltpu.sync_copy(acc.at[pl.ds(lt * zb, zb)],
                        o_hbm.at[pl.ds(c * acc_r + lt * zb, zb)])

    return k(ef, idx2, zeros)


def _pad_rows(x, n):
    return jnp.pad(x, ((0, n - x.shape[0]), (0, 0)))


def _pad_idx(x, n, fill):
    return jnp.pad(x.astype(jnp.int32), (0, n - x.shape[0]),
                   constant_values=fill)


def _mlp_args(p):
    r = lambda v: v.reshape(1, -1)
    return (p["w1"], r(p["b1"]), p["w2"], r(p["b2"]),
            r(p["g"]), r(p["be"]))


def kernel(grid_nfeat, mesh_nfeat, g2m_efeat, mesh_efeat, m2g_efeat, params,
           g2m_src, g2m_dst, mesh_src, mesh_dst, m2g_src, m2g_dst):
    f32 = jnp.float32
    zeros = jnp.zeros((1024, 128), f32)

    xr = grid_nfeat[0].reshape(IN_GRID, NGD).astype(f32)
    zc = jnp.zeros((IN_GRID, 2), f32)
    xt = jnp.concatenate([xr[:, :RD], zc, xr[:, RD:], zc], axis=1)
    mn = _pad_rows(mesh_nfeat.astype(f32), NM)
    eg_in = _pad_rows(g2m_efeat.astype(f32), EG)
    em_in = _pad_rows(mesh_efeat.astype(f32), EM)
    ep_in = _pad_rows(m2g_efeat.astype(f32), EP)

    def to_layout(i):
        i = i.astype(jnp.int32)
        return jnp.where(i < RD, i, i + 2)

    g2m_src_p = _pad_idx(to_layout(g2m_src), EG, 0)
    mesh_src_p = _pad_idx(mesh_src, EM, 0)
    m2g_src_p = _pad_idx(m2g_src, EP, 0)
    g2m_dst_p = _pad_idx(g2m_dst, EG, 0)
    mesh_dst_p = _pad_idx(mesh_dst, EM, 0)
    m2g_dst_p = _pad_idx(to_layout(m2g_dst), EP, 0)

    def dbl(i):
        return jnp.stack([2 * i, 2 * i + 1], axis=-1).reshape(-1)

    g2m_dst_sc = dbl(_pad_idx(g2m_dst, EG, NMD))
    mesh_dst_sc = dbl(_pad_idx(mesh_dst, EM, NMD))
    mesh_src2 = dbl(mesh_src_p)
    mesh_dst2 = dbl(mesh_dst_p)
    gl = _pad_idx(to_layout(m2g_dst), EP, GB1 + RD)

    def grid_idx(q):
        secs = []
        for r in (2 * q, 2 * q + 1):
            base = r * GQ
            secs.append(jnp.where((gl >= base) & (gl < base + GQ),
                                  gl - base, GQ))
        return dbl(jnp.concatenate(secs))

    pp = params

    w1s = pp["g2m_edge_mlp"]["w1"]
    g0, a_g2m = _grid_encoder(xt, *_mlp_args(pp["grid_enc"]), w1s[H:2 * H])
    m0, b_g2m = _mesh_encoder(mn, *_mlp_args(pp["mesh_enc"]), w1s[2 * H:])
    e_g2m = _edge_encoder(eg_in, *_mlp_args(pp["g2m_edge_enc"]))
    e_mesh = _edge_encoder(em_in, *_mlp_args(pp["mesh_edge_enc"]))
    e_m2g = _edge_encoder(ep_in, *_mlp_args(pp["m2g_edge_enc"]))

    pre0 = _sc_gather_add(a_g2m.reshape(2 * NG, 128),
                          b_g2m.reshape(2 * NM, 128),
                          dbl(g2m_src_p), dbl(g2m_dst_p), 208).reshape(EG, H)
    pe = pp["g2m_edge_mlp"]
    ef = _edge_mlp(e_g2m, pre0, pe["w1"][:H], *_mlp_args(pe)[1:])
    parts = _sc_scatter_mesh(ef.reshape(2 * EG, 128), g2m_dst_sc,
                             zeros, 208).reshape(2 * R_MESH, H)
    pn = pp["g2m_node_mlp"]
    w1e0 = pp["proc"][0]["edge"]["w1"]
    m1, a_p, b_p = _node_mlp_mesh(m0, parts, pn["w1"][:H], pn["w1"][H:],
                                  *_mlp_args(pn)[1:], w1e0[H:2 * H],
                                  w1e0[2 * H:])
    pg = pp["g2m_grid_mlp"]
    g1, b_m2g = _grid_mlp(g0, *_mlp_args(pg), pp["m2g_edge_mlp"]["w1"][2 * H:])

    m = m1
    for i in range(N_PROC):
        pre0 = _sc_gather_add(a_p.reshape(2 * NM, 128),
                              b_p.reshape(2 * NM, 128),
                              mesh_src2, mesh_dst2, 256).reshape(EM, H)
        pe = pp["proc"][i]["edge"]
        ef = _edge_mlp(e_mesh, pre0, pe["w1"][:H], *_mlp_args(pe)[1:])
        e_mesh = ef
        parts = _sc_scatter_mesh(ef.reshape(2 * EM, 128), mesh_dst_sc,
                                 zeros, 256).reshape(2 * R_MESH, H)
        pn = pp["proc"][i]["node"]
        if i + 1 < N_PROC:
            w1n = pp["proc"][i + 1]["edge"]["w1"]
            wa, wb = w1n[H:2 * H], w1n[2 * H:]
        else:
            w1n = pp["m2g_edge_mlp"]["w1"]
            wa, wb = w1n[H:2 * H], w1n[H:2 * H]
        m, a_p, b_p = _node_mlp_mesh(m, parts, pn["w1"][:H], pn["w1"][H:],
                                     *_mlp_args(pn)[1:], wa, wb)

    pre0 = _sc_gather_add(a_p.reshape(2 * NM, 128),
                          b_m2g.reshape(2 * NG, 128),
                          dbl(m2g_src_p), dbl(m2g_dst_p), 256).reshape(EP, H)
    pe = pp["m2g_edge_mlp"]
    ef = _edge_mlp(e_m2g, pre0, pe["w1"][:H], *_mlp_args(pe)[1:])
    ef2 = ef.reshape(2 * EP, 128)
    agg = jnp.concatenate([
        _sc_scatter_grid(ef2, grid_idx(0), zeros, 256),
        _sc_scatter_grid(ef2, grid_idx(1), zeros, 256),
    ]).reshape(NG, H)
    pn = pp["m2g_node_mlp"]
    pd = pp["decoder"]
    out = _grid_node_decoder(g1, agg, pn["w1"][:H], pn["w1"][H:],
                             *_mlp_args(pn)[1:],
                             pd["w1"], pd["b1"].reshape(1, -1),
                             pd["w2"], pd["b2"].reshape(1, -1))

    out = jnp.concatenate([out[:RD], out[GB1:GB1 + RD]])
    return out.T.reshape(1, OUT_GRID, RES_H, RES_W)
